# baseline (speedup 1.0000x reference)
import functools

import jax
import jax.numpy as jnp
from jax import lax
from jax.experimental import pallas as pl
from jax.experimental.pallas import tpu as pltpu
from jax.experimental.pallas import tpu_sc as plsc

F = 26
V = 100000
D = 32
NC = 2
NS = 16
L = 16
NW = NC * NS

NB = 64
RPC = NB * F
G = RPC // 128


def _rsqrt_nr(x):
    i = plsc.bitcast(x, jnp.int32)
    i = jnp.int32(0x5F3759DF) - (i >> 1)
    y = plsc.bitcast(i, jnp.float32)
    for _ in range(3):
        y = y * (1.5 - 0.5 * x * y * y)
    return y


@functools.lru_cache(maxsize=None)
def _build(B):
    assert (B * F) % (128 * G * NW) == 0
    chunks = B // (NB * NW)
    blocks_per_worker = chunks * G

    mesh = plsc.VectorSubcoreMesh(
        core_axis_name="c", subcore_axis_name="s",
        num_cores=NC, num_subcores=NS)

    @functools.partial(
        pl.kernel,
        out_type=jax.ShapeDtypeStruct((B * F, D), jnp.float32),
        mesh=mesh,
        compiler_params=pltpu.CompilerParams(
            needs_layout_passes=False, use_tc_tiling_on_sc=False),
        scratch_types=[
            pltpu.VMEM((RPC,), jnp.int32),
            pltpu.VMEM((G, 128), jnp.int32),
            pltpu.VMEM((RPC, D), jnp.float32),
            pltpu.SemaphoreType.DMA,
        ],
    )
    def emb_kernel(metas_hbm, tables_hbm, out_hbm, mstage, midx, rows, gsem):
        wid = lax.axis_index("s") * NC + lax.axis_index("c")

        def chunk(i, _):
            blk = wid * blocks_per_worker + i * G
            rbase = blk * 128

            pltpu.sync_copy(metas_hbm.at[pl.ds(rbase, RPC)], mstage)

            def fix(j, _):
                for l in range(128 // L):
                    pos = lax.iota(jnp.int32, L) + (j * 128 + l * L)
                    off = (pos % F) * V
                    midx[j, pl.ds(l * L, L)] = (
                        mstage[pl.ds(j * 128 + l * L, L)] + off)
                return 0
            lax.fori_loop(0, G, fix, 0)

            copies = [
                pltpu.make_async_copy(
                    tables_hbm.at[midx.at[j]],
                    rows.at[pl.ds(j * 128, 128)],
                    gsem)
                for j in range(G)
            ]
            for c in copies:
                c.start()
            for c in copies:
                c.wait()

            def norm16(r, _):
                rows16 = r * L + lax.iota(jnp.int32, L)
                vals = []
                acc = jnp.zeros((L,), jnp.float32)
                for d in range(D):
                    dv = jnp.full((L,), d, jnp.int32)
                    v = plsc.load_gather(rows, [rows16, dv])
                    vals.append(v)
                    acc = acc + v * v
                inv = _rsqrt_nr(acc)
                inv = jnp.where(acc <= 1e-16, 1.0, inv)
                for d in range(D):
                    dv = jnp.full((L,), d, jnp.int32)
                    plsc.store_scatter(rows, [rows16, dv], vals[d] * inv)
                return 0
            lax.fori_loop(0, RPC // L, norm16, 0)

            pltpu.sync_copy(rows, out_hbm.at[pl.ds(rbase, RPC)])
            return 0

        lax.fori_loop(0, chunks, chunk, 0)

    return emb_kernel


def kernel(metas, tables):
    B = metas.shape[0]
    metas_flat = metas.reshape(-1)
    tables_flat = tables.reshape(F * V, D)
    out = _build(B)(metas_flat, tables_flat)
    return out.reshape(B, F * D)


# speedup vs baseline: 1.0004x; 1.0004x over previous
import functools

import jax
import jax.numpy as jnp
from jax import lax
from jax.experimental import pallas as pl
from jax.experimental.pallas import tpu as pltpu
from jax.experimental.pallas import tpu_sc as plsc

F = 26
V = 100000
D = 32
NC = 2
NS = 16
L = 16
NW = NC * NS

NB = 64
RPC = NB * F
G = RPC // 128


def _rsqrt_nr(x):
    i = plsc.bitcast(x, jnp.int32)
    i = jnp.int32(0x5F3759DF) - (i >> 1)
    y = plsc.bitcast(i, jnp.float32)
    for _ in range(3):
        y = y * (1.5 - 0.5 * x * y * y)
    return y


@functools.lru_cache(maxsize=None)
def _build(B):
    assert (B * F) % (128 * G * NW) == 0
    chunks = B // (NB * NW)
    blocks_per_worker = chunks * G

    mesh = plsc.VectorSubcoreMesh(
        core_axis_name="c", subcore_axis_name="s",
        num_cores=NC, num_subcores=NS)

    @functools.partial(
        pl.kernel,
        out_type=jax.ShapeDtypeStruct((B * F, D), jnp.float32),
        mesh=mesh,
        compiler_params=pltpu.CompilerParams(
            needs_layout_passes=False, use_tc_tiling_on_sc=False),
        scratch_types=[
            pltpu.VMEM((RPC,), jnp.int32),
            pltpu.VMEM((G, 128), jnp.int32),
            pltpu.VMEM((RPC, D), jnp.float32),
            pltpu.SemaphoreType.DMA,
        ],
    )
    def emb_kernel(metas_hbm, tables_hbm, out_hbm, mstage, midx, rows, gsem):
        wid = lax.axis_index("s") * NC + lax.axis_index("c")

        def chunk(i, _):
            blk = wid * blocks_per_worker + i * G
            rbase = blk * 128

            pltpu.sync_copy(metas_hbm.at[pl.ds(rbase, RPC)], mstage)

            def fix(j, _):
                for l in range(128 // L):
                    pos = lax.iota(jnp.int32, L) + (j * 128 + l * L)
                    off = (pos % F) * V
                    midx[j, pl.ds(l * L, L)] = (
                        mstage[pl.ds(j * 128 + l * L, L)] + off)
                return 0
            lax.fori_loop(0, G, fix, 0)

            copies = [
                pltpu.make_async_copy(
                    tables_hbm.at[midx.at[j]],
                    rows.at[pl.ds(j * 128, 128)],
                    gsem)
                for j in range(G)
            ]
            for c in copies:
                c.start()
            for c in copies:
                c.wait()

            def norm16(r, _):
                rows16 = r * L + lax.iota(jnp.int32, L)
                vals = []
                acc = jnp.zeros((L,), jnp.float32)
                for d in range(D):
                    dv = jnp.full((L,), d, jnp.int32)
                    v = plsc.load_gather(rows, [rows16, dv])
                    vals.append(v)
                    acc = acc + v * v
                inv = _rsqrt_nr(acc)
                inv = jnp.where(acc <= 1e-16, 1.0, inv)
                for d in range(D):
                    dv = jnp.full((L,), d, jnp.int32)
                    plsc.store_scatter(rows, [rows16, dv], vals[d] * inv)
                return 0
            lax.fori_loop(0, RPC // L, norm16, 0)

            pltpu.sync_copy(rows, out_hbm.at[pl.ds(rbase, RPC)])
            return 0

        lax.fori_loop(0, chunks, chunk, 0)

    return emb_kernel


def kernel(metas, tables):
    B = metas.shape[0]
    metas_flat = metas.reshape(-1)
    tables_1d = jax.lax.optimization_barrier(tables.reshape(-1))
    tables_flat = tables_1d.reshape(F * V, D)
    out = _build(B)(metas_flat, tables_flat)
    return out.reshape(B, F * D)


# speedup vs baseline: 1.0624x; 1.0619x over previous
import functools

import jax
import jax.numpy as jnp
from jax import lax
from jax.experimental import pallas as pl
from jax.experimental.pallas import tpu as pltpu
from jax.experimental.pallas import tpu_sc as plsc

F = 26
V = 100000
VP = 100096
D = 32
NC = 2
NS = 16
L = 16
NW = NC * NS

NB = 64
RPC = NB * F
G = RPC // 128

BK = 2944
NJ = VP // BK


def _regroup_body(x_ref, o_ref):
    x = x_ref[0]
    y = x.T
    y2 = y.reshape(BK // 4, 4, D)
    o_ref[...] = jnp.concatenate(
        [y2[:, 0, :], y2[:, 1, :], y2[:, 2, :], y2[:, 3, :]], axis=-1)


_regroup = pl.pallas_call(
    _regroup_body,
    grid=(F, NJ),
    in_specs=[pl.BlockSpec((1, D, BK), lambda f, j: (f, 0, j))],
    out_specs=pl.BlockSpec((BK // 4, 128), lambda f, j: (f * NJ + j, 0)),
    out_shape=jax.ShapeDtypeStruct((F * VP * D // 128, 128), jnp.float32),
)


def _rsqrt_nr(x):
    i = plsc.bitcast(x, jnp.int32)
    i = jnp.int32(0x5F3759DF) - (i >> 1)
    y = plsc.bitcast(i, jnp.float32)
    for _ in range(3):
        y = y * (1.5 - 0.5 * x * y * y)
    return y


@functools.lru_cache(maxsize=None)
def _build(B):
    assert (B * F) % (128 * G * NW) == 0
    chunks = B // (NB * NW)
    blocks_per_worker = chunks * G

    mesh = plsc.VectorSubcoreMesh(
        core_axis_name="c", subcore_axis_name="s",
        num_cores=NC, num_subcores=NS)

    @functools.partial(
        pl.kernel,
        out_type=jax.ShapeDtypeStruct((B * F, D), jnp.float32),
        mesh=mesh,
        compiler_params=pltpu.CompilerParams(
            needs_layout_passes=False, use_tc_tiling_on_sc=False),
        scratch_types=[
            pltpu.VMEM((RPC,), jnp.int32),
            pltpu.VMEM((G, 128), jnp.int32),
            pltpu.VMEM((RPC, D), jnp.float32),
            pltpu.SemaphoreType.DMA,
        ],
    )
    def emb_kernel(metas_hbm, tables_hbm, out_hbm, mstage, midx, rows, gsem):
        wid = lax.axis_index("s") * NC + lax.axis_index("c")

        def chunk(i, _):
            blk = wid * blocks_per_worker + i * G
            rbase = blk * 128

            pltpu.sync_copy(metas_hbm.at[pl.ds(rbase, RPC)], mstage)

            def fix(j, _):
                for l in range(128 // L):
                    pos = lax.iota(jnp.int32, L) + (j * 128 + l * L)
                    off = (pos % F) * VP
                    midx[j, pl.ds(l * L, L)] = (
                        mstage[pl.ds(j * 128 + l * L, L)] + off)
                return 0
            lax.fori_loop(0, G, fix, 0)

            copies = [
                pltpu.make_async_copy(
                    tables_hbm.at[midx.at[j]],
                    rows.at[pl.ds(j * 128, 128)],
                    gsem)
                for j in range(G)
            ]
            for c in copies:
                c.start()
            for c in copies:
                c.wait()

            def norm16(r, _):
                rows16 = r * L + lax.iota(jnp.int32, L)
                vals = []
                acc = jnp.zeros((L,), jnp.float32)
                for d in range(D):
                    dv = jnp.full((L,), d, jnp.int32)
                    v = plsc.load_gather(rows, [rows16, dv])
                    vals.append(v)
                    acc = acc + v * v
                inv = _rsqrt_nr(acc)
                inv = jnp.where(acc <= 1e-16, 1.0, inv)
                for d in range(D):
                    dv = jnp.full((L,), d, jnp.int32)
                    plsc.store_scatter(rows, [rows16, dv], vals[d] * inv)
                return 0
            lax.fori_loop(0, RPC // L, norm16, 0)

            pltpu.sync_copy(rows, out_hbm.at[pl.ds(rbase, RPC)])
            return 0

        lax.fori_loop(0, chunks, chunk, 0)

    return emb_kernel


def kernel(metas, tables):
    B = metas.shape[0]
    metas_flat = metas.reshape(-1)
    tt = tables.transpose(0, 2, 1)
    dense128 = _regroup(tt)
    tables_flat = dense128.reshape(-1).reshape(F * VP, D)
    out = _build(B)(metas_flat, tables_flat)
    return out.reshape(B, F * D)
